# baseline (speedup 1.0000x reference)
import jax
import jax.numpy as jnp
from jax import lax
from jax.experimental import pallas as pl
from jax.experimental.pallas import tpu as pltpu
from jax.experimental.pallas import tpu_sc as plsc

NCLASS = 1000
N = 16384
NUM_CORES = 2
NUM_SUBCORES = 16
NW = NUM_CORES * NUM_SUBCORES
RPW = N // NW
BATCH = 32
NB = RPW // BATCH
LANES = 16


def _sc_onehot(x_hbm, zeros_hbm, out_hbm, x_v, buf0, buf1, sem0, sem1):
    wid = lax.axis_index("s") * NUM_CORES + lax.axis_index("c")
    base = wid * RPW

    pltpu.sync_copy(x_hbm.at[pl.ds(base, RPW)], x_v)
    pltpu.sync_copy(zeros_hbm, buf0)
    pltpu.sync_copy(zeros_hbm, buf1)

    bufs = (buf0, buf1)
    sems = (sem0, sem1)
    ones = jnp.ones((LANES,), jnp.float32)
    zvals = jnp.zeros((LANES,), jnp.float32)
    row_iota = lax.iota(jnp.int32, LANES)

    copies = [None] * NB
    for b in range(NB):
        buf = bufs[b % 2]
        if b >= 2:
            copies[b - 2].wait()
            for j in range(BATCH // LANES):
                r16 = row_iota + (j * LANES)
                c16 = x_v[pl.ds((b - 2) * BATCH + j * LANES, LANES)]
                plsc.store_scatter(buf, [r16, c16], zvals)
        for j in range(BATCH // LANES):
            r16 = row_iota + (j * LANES)
            c16 = x_v[pl.ds(b * BATCH + j * LANES, LANES)]
            plsc.store_scatter(buf, [r16, c16], ones)
        cp = pltpu.make_async_copy(
            buf, out_hbm.at[pl.ds(base + b * BATCH, BATCH)], sems[b % 2]
        )
        cp.start()
        copies[b] = cp
    copies[NB - 2].wait()
    copies[NB - 1].wait()


def kernel(x):
    x = x.astype(jnp.int32)
    zeros = jnp.zeros((BATCH, NCLASS), jnp.float32)
    mesh = plsc.VectorSubcoreMesh(
        core_axis_name="c", subcore_axis_name="s",
        num_cores=NUM_CORES, num_subcores=NUM_SUBCORES,
    )
    call = pl.kernel(
        _sc_onehot,
        out_type=jax.ShapeDtypeStruct((N, NCLASS), jnp.float32),
        mesh=mesh,
        scratch_types=[
            pltpu.VMEM((RPW,), jnp.int32),
            pltpu.VMEM((BATCH, NCLASS), jnp.float32),
            pltpu.VMEM((BATCH, NCLASS), jnp.float32),
            pltpu.SemaphoreType.DMA,
            pltpu.SemaphoreType.DMA,
        ],
        compiler_params=pltpu.CompilerParams(
            use_tc_tiling_on_sc=False, needs_layout_passes=False
        ),
    )
    return call(x, zeros)


# speedup vs baseline: 1.5073x; 1.5073x over previous
import jax
import jax.numpy as jnp
from jax import lax
from jax.experimental import pallas as pl
from jax.experimental.pallas import tpu as pltpu
from jax.experimental.pallas import tpu_sc as plsc

NCLASS = 1000
N = 16384
NUM_CORES = 2
NUM_SUBCORES = 16
NW = NUM_CORES * NUM_SUBCORES
RPW = N // NW
BATCH = 32
NB = RPW // BATCH
LANES = 16


def _sc_onehot(x_hbm, zeros_hbm, out_hbm, x_v, buf0, buf1, sem0, sem1):
    wid = lax.axis_index("s") * NUM_CORES + lax.axis_index("c")
    base = wid * RPW

    pltpu.sync_copy(x_hbm.at[pl.ds(base, RPW)], x_v)
    pltpu.sync_copy(zeros_hbm, buf0)
    pltpu.sync_copy(zeros_hbm, buf1)

    bufs = (buf0, buf1)
    sems = (sem0, sem1)
    ones = jnp.ones((LANES,), jnp.float32)
    zvals = jnp.zeros((LANES,), jnp.float32)
    row_iota = lax.iota(jnp.int32, LANES)

    copies = [None] * NB
    for b in range(NB):
        buf = bufs[b % 2]
        if b >= 2:
            copies[b - 2].wait()
            for j in range(BATCH // LANES):
                r16 = row_iota + (j * LANES)
                c16 = x_v[pl.ds((b - 2) * BATCH + j * LANES, LANES)]
                plsc.store_scatter(buf, [r16, c16], zvals)
        for j in range(BATCH // LANES):
            r16 = row_iota + (j * LANES)
            c16 = x_v[pl.ds(b * BATCH + j * LANES, LANES)]
            plsc.store_scatter(buf, [r16, c16], ones)
        cp = pltpu.make_async_copy(
            buf, out_hbm.at[pl.ds(base + b * BATCH, BATCH)], sems[b % 2]
        )
        cp.start()
        copies[b] = cp
    copies[NB - 2].wait()
    copies[NB - 1].wait()


def kernel(x):
    x = x.astype(jnp.int32)
    zeros = jnp.zeros((BATCH, NCLASS), jnp.float32)
    mesh = plsc.VectorSubcoreMesh(
        core_axis_name="c", subcore_axis_name="s",
        num_cores=NUM_CORES, num_subcores=NUM_SUBCORES,
    )
    call = pl.kernel(
        _sc_onehot,
        out_type=jax.ShapeDtypeStruct((N, NCLASS), jnp.float32),
        mesh=mesh,
        scratch_types=[
            pltpu.VMEM((RPW,), jnp.int32),
            pltpu.VMEM((BATCH, NCLASS), jnp.float32),
            pltpu.VMEM((BATCH, NCLASS), jnp.float32),
            pltpu.SemaphoreType.DMA,
            pltpu.SemaphoreType.DMA,
        ],
        compiler_params=pltpu.CompilerParams(
            use_tc_tiling_on_sc=True, needs_layout_passes=False
        ),
    )
    return call(x, zeros)


# speedup vs baseline: 2.8710x; 1.9047x over previous
import jax
import jax.numpy as jnp
from jax import lax
from jax.experimental import pallas as pl
from jax.experimental.pallas import tpu as pltpu
from jax.experimental.pallas import tpu_sc as plsc

NCLASS = 1000
N = 16384
NUM_CORES = 2
NUM_SUBCORES = 16
NW = NUM_CORES * NUM_SUBCORES
CPW = N // NW
SLAB = 128
SLABS_PER_W = CPW // SLAB
ROW_CHUNKS = (0, 256, 512, 768, NCLASS)
RH_MAX = 256
LANES = 16


def _sc_onehot_t(x_hbm, zeros_hbm, out_hbm, x_v, buf0, buf1, sem0, sem1):
    wid = lax.axis_index("s") * NUM_CORES + lax.axis_index("c")
    col_base = wid * CPW

    pltpu.sync_copy(x_hbm.at[pl.ds(col_base, CPW)], x_v)
    pltpu.sync_copy(zeros_hbm, buf0)
    pltpu.sync_copy(zeros_hbm, buf1)

    bufs = (buf0, buf1)
    sems = (sem0, sem1)
    ones = jnp.ones((LANES,), jnp.float32)
    zvals = jnp.zeros((LANES,), jnp.float32)
    lane_iota = lax.iota(jnp.int32, LANES)

    chunks = [
        (s, q) for s in range(SLABS_PER_W) for q in range(len(ROW_CHUNKS) - 1)
    ]

    def scatter_chunk(s, q, buf, vals):
        r0 = ROW_CHUNKS[q]
        rh = ROW_CHUNKS[q + 1] - r0
        for j in range(SLAB // LANES):
            xv = x_v[pl.ds(s * SLAB + j * LANES, LANES)]
            col16 = lane_iota + (j * LANES)
            mask = (xv >= r0) & (xv < r0 + rh)
            rloc = jnp.clip(xv - r0, 0, rh - 1)
            plsc.store_scatter(buf, [rloc, col16], vals, mask=mask)

    copies = [None] * len(chunks)
    for ci, (s, q) in enumerate(chunks):
        buf = bufs[ci % 2]
        if ci >= 2:
            copies[ci - 2].wait()
            ps, pq = chunks[ci - 2]
            scatter_chunk(ps, pq, buf, zvals)
        scatter_chunk(s, q, buf, ones)
        r0 = ROW_CHUNKS[q]
        rh = ROW_CHUNKS[q + 1] - r0
        cp = pltpu.make_async_copy(
            buf.at[pl.ds(0, rh)],
            out_hbm.at[pl.ds(r0, rh), pl.ds(col_base + s * SLAB, SLAB)],
            sems[ci % 2],
        )
        cp.start()
        copies[ci] = cp
    copies[-2].wait()
    copies[-1].wait()


def kernel(x):
    x = x.astype(jnp.int32)
    zeros = jnp.zeros((RH_MAX, SLAB), jnp.float32)
    mesh = plsc.VectorSubcoreMesh(
        core_axis_name="c", subcore_axis_name="s",
        num_cores=NUM_CORES, num_subcores=NUM_SUBCORES,
    )
    call = pl.kernel(
        _sc_onehot_t,
        out_type=jax.ShapeDtypeStruct((NCLASS, N), jnp.float32),
        mesh=mesh,
        scratch_types=[
            pltpu.VMEM((CPW,), jnp.int32),
            pltpu.VMEM((RH_MAX, SLAB), jnp.float32),
            pltpu.VMEM((RH_MAX, SLAB), jnp.float32),
            pltpu.SemaphoreType.DMA,
            pltpu.SemaphoreType.DMA,
        ],
        compiler_params=pltpu.CompilerParams(
            use_tc_tiling_on_sc=True, needs_layout_passes=False
        ),
    )
    return call(x, zeros).T


# speedup vs baseline: 7.8444x; 2.7322x over previous
import jax
import jax.numpy as jnp
from jax import lax
from jax.experimental import pallas as pl

NCLASS = 1000
N = 16384
BLK = 2048


def _onehot(x_ref, o_ref):
    xb = x_ref[...]
    rows = lax.broadcasted_iota(jnp.int32, (NCLASS, BLK), 0)
    o_ref[...] = jnp.where(rows == xb[None, :], 1.0, 0.0).astype(jnp.float32)


def kernel(x):
    x = x.astype(jnp.int32)
    z = pl.pallas_call(
        _onehot,
        out_shape=jax.ShapeDtypeStruct((NCLASS, N), jnp.float32),
        grid=(N // BLK,),
        in_specs=[pl.BlockSpec((BLK,), lambda i: (i,))],
        out_specs=pl.BlockSpec((NCLASS, BLK), lambda i: (0, i)),
    )(x)
    return z.T


# speedup vs baseline: 8.2571x; 1.0526x over previous
import jax
import jax.numpy as jnp
from jax import lax
from jax.experimental import pallas as pl

NCLASS = 1000
N = 16384
BLK = 1024


def _onehot(x_ref, o_ref):
    xb = x_ref[...]
    rows = lax.broadcasted_iota(jnp.int32, (NCLASS, BLK), 0)
    o_ref[...] = jnp.where(rows == xb[None, :], 1.0, 0.0).astype(jnp.float32)


def kernel(x):
    x = x.astype(jnp.int32)
    z = pl.pallas_call(
        _onehot,
        out_shape=jax.ShapeDtypeStruct((NCLASS, N), jnp.float32),
        grid=(N // BLK,),
        in_specs=[pl.BlockSpec((BLK,), lambda i: (i,))],
        out_specs=pl.BlockSpec((NCLASS, BLK), lambda i: (0, i)),
    )(x)
    return z.T
